# Initial kernel scaffold; baseline (speedup 1.0000x reference)
import jax
import jax.numpy as jnp
from jax.experimental import pallas as pl


def kernel(x, edge_index, batch, Wx, bx, Wl0, Wr0, att0, b0, Wl1, Wr1, att1, b1, Wp, bp):
    raise NotImplementedError("write your pallas kernel here")


# speedup vs baseline: 12.7207x; 12.7207x over previous
import dataclasses

import jax
import jax.numpy as jnp
from jax import lax
from jax.experimental import pallas as pl
from jax.experimental.pallas import tpu as pltpu
from jax.experimental.pallas import tpu_sc as plsc

N_NODES = 10000
N_EDGES = 320000
IN_CH = 128
D = 256
C = 32
N_GRAPHS = 64

EB = 80
EDGES_PER_TILE = N_EDGES // 16
NBLK = EDGES_PER_TILE // EB
ROWS_PER_TILE = N_NODES // 16

_HI = jax.lax.Precision.HIGHEST


def _sc_compiler_params():
  cp = pltpu.CompilerParams()
  fields = pltpu.CompilerParams.__dataclass_fields__
  if "needs_layout_passes" in fields:
    cp = dataclasses.replace(cp, needs_layout_passes=False)
  if "use_tc_tiling_on_sc" in fields:
    cp = dataclasses.replace(cp, use_tc_tiling_on_sc=False)
  return cp


def _head_mask():
  jj = lax.broadcasted_iota(jnp.int32, (128, 16), 0)
  hh = lax.broadcasted_iota(jnp.int32, (128, 16), 1)
  return jnp.where(jj // C == hh, 1.0, 0.0).astype(jnp.float32)


def _expand_mask():
  kk = lax.broadcasted_iota(jnp.int32, (16, 128), 0)
  jj = lax.broadcasted_iota(jnp.int32, (16, 128), 1)
  return jnp.where(jj // C == kk, 1.0, 0.0).astype(jnp.float32)


def _project(h, s, wl_ref, wr_ref, attf_ref, glp_refs, grp_refs):
  gl = jnp.dot(h, wl_ref[s], precision=_HI,
               preferred_element_type=jnp.float32)
  gr = jnp.dot(h, wr_ref[s], precision=_HI,
               preferred_element_type=jnp.float32)
  glp_refs[2 * s][...] = gl[:, :64]
  glp_refs[2 * s + 1][...] = gl[:, 64:]
  grp_refs[2 * s][...] = gr[:, :64]
  grp_refs[2 * s + 1][...] = gr[:, 64:]
  pabs = jnp.abs(gl) * jnp.abs(attf_ref[s])[None, :]
  pb = jnp.dot(pabs, _head_mask(), precision=_HI,
               preferred_element_type=jnp.float32)
  return jnp.max(pb, axis=0, keepdims=True)


def _k1_body(x_ref, wx_ref, bx_ref, wl_ref, wr_ref, attf_ref,
             h_ref, glp0, glp1, glp2, glp3, grp0, grp1, grp2, grp3,
             maxp_ref):
  i = pl.program_id(0)
  h = jnp.dot(x_ref[...], wx_ref[...], precision=_HI,
              preferred_element_type=jnp.float32) + bx_ref[...]
  h_ref[...] = h
  glp_refs = [glp0, glp1, glp2, glp3]
  grp_refs = [grp0, grp1, grp2, grp3]
  pms = [_project(h, s, wl_ref, wr_ref, attf_ref, glp_refs, grp_refs)
         for s in range(2)]
  pm = jnp.concatenate(pms, axis=0)

  @pl.when(i == 0)
  def _():
    maxp_ref[...] = jnp.zeros_like(maxp_ref)

  maxp_ref[...] = jnp.maximum(maxp_ref[...], pm)


def _table_out_specs(bn):
  specs = [pl.BlockSpec((bn, D), lambda i: (i, 0))]
  for _ in range(8):
    specs.append(pl.BlockSpec((bn, 64), lambda i: (i, 0)))
  specs.append(pl.BlockSpec((2, 16), lambda i: (0, 0)))
  return specs


def _table_out_shapes():
  shapes = [jax.ShapeDtypeStruct((N_NODES, D), jnp.float32)]
  for _ in range(8):
    shapes.append(jax.ShapeDtypeStruct((N_NODES, 64), jnp.float32))
  shapes.append(jax.ShapeDtypeStruct((2, 16), jnp.float32))
  return shapes


def _run_k1(x, Wx, bxr, Wlp, Wrp, attf):
  bn = 1000
  nb = N_NODES // bn
  return pl.pallas_call(
      _k1_body,
      grid=(nb,),
      in_specs=[
          pl.BlockSpec((bn, IN_CH), lambda i: (i, 0)),
          pl.BlockSpec((IN_CH, D), lambda i: (0, 0)),
          pl.BlockSpec((1, D), lambda i: (0, 0)),
          pl.BlockSpec((2, D, 128), lambda i: (0, 0, 0)),
          pl.BlockSpec((2, D, 128), lambda i: (0, 0, 0)),
          pl.BlockSpec((2, 128), lambda i: (0, 0)),
      ],
      out_specs=_table_out_specs(bn),
      out_shape=_table_out_shapes(),
  )(x, Wx, bxr, Wlp, Wrp, attf)


def _k2_body(h0_ref, num_ref, den_ref, b_ref, wl_ref, wr_ref, attf_ref,
             h_ref, glp0, glp1, glp2, glp3, grp0, grp1, grp2, grp3,
             maxp_ref):
  i = pl.program_id(0)
  em = _expand_mask()
  bn = h0_ref.shape[0]
  zpad = jnp.zeros((bn, 12), jnp.float32)
  combs = []
  for s in range(2):
    d16 = jnp.concatenate(
        [den_ref[2 * s][:, :2], den_ref[2 * s + 1][:, :2], zpad], axis=1)
    dexp = jnp.dot(d16, em, precision=_HI,
                   preferred_element_type=jnp.float32) + 1e-16
    nums = jnp.concatenate([num_ref[2 * s], num_ref[2 * s + 1]], axis=1)
    combs.append(nums / dexp)
  h = h0_ref[...] + jnp.concatenate(combs, axis=1) + b_ref[...]
  h_ref[...] = h
  glp_refs = [glp0, glp1, glp2, glp3]
  grp_refs = [grp0, grp1, grp2, grp3]
  pms = [_project(h, s, wl_ref, wr_ref, attf_ref, glp_refs, grp_refs)
         for s in range(2)]
  pm = jnp.concatenate(pms, axis=0)

  @pl.when(i == 0)
  def _():
    maxp_ref[...] = jnp.zeros_like(maxp_ref)

  maxp_ref[...] = jnp.maximum(maxp_ref[...], pm)


def _run_k2(h0, num, den, br, Wlp, Wrp, attf):
  bn = 1000
  nb = N_NODES // bn
  return pl.pallas_call(
      _k2_body,
      grid=(nb,),
      in_specs=[
          pl.BlockSpec((bn, D), lambda i: (i, 0)),
          pl.BlockSpec((4, bn, 64), lambda i: (0, i, 0)),
          pl.BlockSpec((4, bn, 16), lambda i: (0, i, 0)),
          pl.BlockSpec((1, D), lambda i: (0, 0)),
          pl.BlockSpec((2, D, 128), lambda i: (0, 0, 0)),
          pl.BlockSpec((2, D, 128), lambda i: (0, 0, 0)),
          pl.BlockSpec((2, 128), lambda i: (0, 0)),
      ],
      out_specs=_table_out_specs(bn),
      out_shape=_table_out_shapes(),
  )(h0, num, den, br, Wlp, Wrp, attf)


def _edge_body(gl_p0, gl_p1, gl_p2, gl_p3, gr_p0, gr_p1, gr_p2, gr_p3,
               src_hbm, dst_hbm, att_hbm, maxp_hbm,
               num_hbm, den_hbm,
               sidx, didx, glb, grb, msgb, denb, attb, mpb, zb, zbd,
               num_sp, den_sp):
  cid = lax.axis_index("c")
  tid = lax.axis_index("s")

  pltpu.sync_copy(att_hbm.at[cid], attb)
  pltpu.sync_copy(maxp_hbm.at[cid], mpb)

  zv = jnp.zeros((16,), jnp.float32)

  @pl.loop(0, 125)
  def _(r):
    for j in range(4):
      zb[r, pl.ds(16 * j, 16)] = zv
    zbd[r, :] = zv

  att_v = [attb[k, :] for k in range(8)]
  atta_v = [jnp.abs(attb[k, :]) for k in range(8)]
  mpv = mpb[:]
  lanes = lax.iota(jnp.int32, 16)
  oh_v = [jnp.where(lanes == j, 1.0, 0.0).astype(jnp.float32)
          for j in range(2)]

  gl_parts = [(gl_p0, gr_p0), (gl_p1, gr_p1), (gl_p2, gr_p2),
              (gl_p3, gr_p3)]

  for rnd in range(2):
    for k in range(5):
      base = tid * ROWS_PER_TILE + k * 125
      pltpu.sync_copy(zb, num_sp.at[pl.ds(base, 125)])
      pltpu.sync_copy(zbd, den_sp.at[pl.ds(base, 125)])

    plsc.subcore_barrier()

    mph = [mpv[2 * rnd + j] for j in range(2)]
    av = [att_v[2 * (2 * rnd + j) + q] for j in range(2) for q in range(2)]
    aav = [atta_v[2 * (2 * rnd + j) + q] for j in range(2) for q in range(2)]

    @pl.loop(0, NBLK)
    def _(blk):
      base = tid * EDGES_PER_TILE + blk * EB
      pltpu.sync_copy(src_hbm.at[pl.ds(base, EB)], sidx)
      pltpu.sync_copy(dst_hbm.at[pl.ds(base, EB)], didx)

      @pl.when(cid == 0)
      def _():
        glr, grr = gl_parts[rnd]
        pltpu.sync_copy(glr.at[sidx], glb)
        pltpu.sync_copy(grr.at[didx], grb)

      @pl.when(cid == 1)
      def _():
        glr, grr = gl_parts[2 + rnd]
        pltpu.sync_copy(glr.at[sidx], glb)
        pltpu.sync_copy(grr.at[didx], grb)

      @pl.loop(0, EB)
      def _(e):
        denv = jnp.zeros((16,), jnp.float32)
        for j in range(2):
          acc = None
          gs = []
          for q in range(2):
            off = C * j + 16 * q
            g = glb[e, pl.ds(off, 16)]
            r = grb[e, pl.ds(off, 16)]
            gs.append(g)
            t = g + r
            l = jnp.maximum(t, 0.2 * t)
            sterm = l * av[2 * j + q] - jnp.abs(r) * aav[2 * j + q]
            acc = sterm if q == 0 else acc + sterm
          lg = jnp.sum(acc) - mph[j]
          ex = jnp.exp(lax.broadcast(lg, (16,)))
          for q in range(2):
            off = C * j + 16 * q
            msgb[e, pl.ds(off, 16)] = gs[q] * ex
          denv = denv + ex * oh_v[j]
        denb[e, :] = denv

      pltpu.sync_copy(msgb, num_sp.at[didx], add=True)
      pltpu.sync_copy(denb, den_sp.at[didx], add=True)

    plsc.subcore_barrier()

    p = 2 * cid + rnd

    @pl.when(tid == 0)
    def _():
      pltpu.sync_copy(num_sp, num_hbm.at[p])
      pltpu.sync_copy(den_sp, den_hbm.at[p])

    plsc.subcore_barrier()


def _run_edge(gl_parts, gr_parts, src, dst, attsc, maxp):
  mesh = plsc.VectorSubcoreMesh(core_axis_name="c", subcore_axis_name="s")
  f = pl.kernel(
      _edge_body,
      out_type=[
          jax.ShapeDtypeStruct((4, N_NODES, 64), jnp.float32),
          jax.ShapeDtypeStruct((4, N_NODES, 16), jnp.float32),
      ],
      mesh=mesh,
      scratch_types=[
          pltpu.VMEM((EB,), jnp.int32),
          pltpu.VMEM((EB,), jnp.int32),
          pltpu.VMEM((EB, 64), jnp.float32),
          pltpu.VMEM((EB, 64), jnp.float32),
          pltpu.VMEM((EB, 64), jnp.float32),
          pltpu.VMEM((EB, 16), jnp.float32),
          pltpu.VMEM((8, 16), jnp.float32),
          pltpu.VMEM((16,), jnp.float32),
          pltpu.VMEM((125, 64), jnp.float32),
          pltpu.VMEM((125, 16), jnp.float32),
          pltpu.VMEM_SHARED((N_NODES, 64), jnp.float32),
          pltpu.VMEM_SHARED((N_NODES, 16), jnp.float32),
      ],
      compiler_params=_sc_compiler_params(),
  )
  return f(*gl_parts, *gr_parts, src, dst, attsc, maxp)


def _pool_body(h1_hbm, num_hbm, den_hbm, b_hbm, batch_hbm,
               hg_hbm, cnt_hbm,
               hb, nbufs0, nbufs1, nbufs2, nbufs3, dbufs0, dbufs1, dbufs2,
               dbufs3, h2b, bb, bidx, onesb, zb8, zb16,
               hg_sp, cnt_sp):
  cid = lax.axis_index("c")
  tid = lax.axis_index("s")

  pltpu.sync_copy(b_hbm, bb)

  zv = jnp.zeros((16,), jnp.float32)
  ov = jnp.ones((16,), jnp.float32)
  for r in range(8):
    for j in range(16):
      zb8[r, pl.ds(16 * j, 16)] = zv
    zb16[r, :] = zv
    onesb[r, :] = ov

  @pl.when(tid == 0)
  def _():
    for k in range(8):
      pltpu.sync_copy(zb8, hg_sp.at[pl.ds(8 * k, 8)])
      pltpu.sync_copy(zb16, cnt_sp.at[pl.ds(8 * k, 8)])

  plsc.subcore_barrier()

  b_v = [bb[j, :] for j in range(16)]
  row0 = cid * 5000 + tid * 312
  nbufs = [nbufs0, nbufs1, nbufs2, nbufs3]
  dbufs = [dbufs0, dbufs1, dbufs2, dbufs3]

  def do_block(base):
    pltpu.sync_copy(h1_hbm.at[pl.ds(base, 8)], hb)
    for p in range(4):
      pltpu.sync_copy(num_hbm.at[p, pl.ds(base, 8)], nbufs[p])
      pltpu.sync_copy(den_hbm.at[p, pl.ds(base, 8)], dbufs[p])
    pltpu.sync_copy(batch_hbm.at[pl.ds(base, 8)], bidx)
    for r in range(8):
      for p in range(4):
        recv = 1.0 / (dbufs[p][r, :] + 1e-16)
        for j in range(2):
          rec = recv[j]
          for q in range(2):
            off = 64 * p + C * j + 16 * q
            noff = C * j + 16 * q
            v = hb[r, pl.ds(off, 16)] + nbufs[p][r, pl.ds(noff, 16)] * rec \
                + b_v[off // 16]
            h2b[r, pl.ds(off, 16)] = v
    pltpu.sync_copy(h2b, hg_sp.at[bidx], add=True)
    pltpu.sync_copy(onesb, cnt_sp.at[bidx], add=True)

  @pl.loop(0, 39)
  def _(blk):
    do_block(row0 + blk * 8)

  @pl.when(tid == 15)
  def _():
    do_block(row0 + 312)

  plsc.subcore_barrier()

  @pl.when(tid == 0)
  def _():
    pltpu.sync_copy(hg_sp, hg_hbm.at[cid])
    pltpu.sync_copy(cnt_sp, cnt_hbm.at[cid])


def _run_pool(h1, num, den, bsc, batch):
  mesh = plsc.VectorSubcoreMesh(core_axis_name="c", subcore_axis_name="s")
  f = pl.kernel(
      _pool_body,
      out_type=[
          jax.ShapeDtypeStruct((2, N_GRAPHS, D), jnp.float32),
          jax.ShapeDtypeStruct((2, N_GRAPHS, 16), jnp.float32),
      ],
      mesh=mesh,
      scratch_types=[
          pltpu.VMEM((8, D), jnp.float32),
          pltpu.VMEM((8, 64), jnp.float32),
          pltpu.VMEM((8, 64), jnp.float32),
          pltpu.VMEM((8, 64), jnp.float32),
          pltpu.VMEM((8, 64), jnp.float32),
          pltpu.VMEM((8, 16), jnp.float32),
          pltpu.VMEM((8, 16), jnp.float32),
          pltpu.VMEM((8, 16), jnp.float32),
          pltpu.VMEM((8, 16), jnp.float32),
          pltpu.VMEM((8, D), jnp.float32),
          pltpu.VMEM((16, 16), jnp.float32),
          pltpu.VMEM((8,), jnp.int32),
          pltpu.VMEM((8, 16), jnp.float32),
          pltpu.VMEM((8, D), jnp.float32),
          pltpu.VMEM((8, 16), jnp.float32),
          pltpu.VMEM_SHARED((N_GRAPHS, D), jnp.float32),
          pltpu.VMEM_SHARED((N_GRAPHS, 16), jnp.float32),
      ],
      compiler_params=_sc_compiler_params(),
  )
  return f(h1, num, den, bsc, batch)


def _k3_body(hg_ref, cnt_ref, wp_ref, bp_ref, out_ref):
  hs = hg_ref[0] + hg_ref[1]
  ct = cnt_ref[0, :, 0:1] + cnt_ref[1, :, 0:1]
  hgm = hs / jnp.maximum(ct, 1.0)
  out_ref[...] = jnp.dot(hgm, wp_ref[...], precision=_HI,
                         preferred_element_type=jnp.float32) + bp_ref[...]


def _run_final(hgsum, counts, Wp, bpr):
  return pl.pallas_call(
      _k3_body,
      out_shape=jax.ShapeDtypeStruct((N_GRAPHS, 1), jnp.float32),
  )(hgsum, counts, Wp, bpr)


def kernel(x, edge_index, batch, Wx, bx, Wl0, Wr0, att0, b0, Wl1, Wr1, att1,
           b1, Wp, bp):
  src = edge_index[0].astype(jnp.int32)
  dst = edge_index[1].astype(jnp.int32)
  batch32 = batch.astype(jnp.int32)

  def parts(W):
    return jnp.moveaxis(W.reshape(D, 2, 128), 1, 0)

  Wl0p, Wr0p = parts(Wl0), parts(Wr0)
  Wl1p, Wr1p = parts(Wl1), parts(Wr1)
  att0f = att0.reshape(2, 128)
  att1f = att1.reshape(2, 128)
  att0sc = att0.reshape(2, 8, 16)
  att1sc = att1.reshape(2, 8, 16)
  bxr = bx.reshape(1, D)
  b0r = b0.reshape(1, D)
  b1sc = b1.reshape(16, 16)
  bpr = bp.reshape(1, 1)

  o1 = _run_k1(x, Wx, bxr, Wl0p, Wr0p, att0f)
  h0, gl0_parts, gr0_parts, maxp0 = o1[0], o1[1:5], o1[5:9], o1[9]
  num0, den0 = _run_edge(gl0_parts, gr0_parts, src, dst, att0sc, maxp0)
  o2 = _run_k2(h0, num0, den0, b0r, Wl1p, Wr1p, att1f)
  h1, gl1_parts, gr1_parts, maxp1 = o2[0], o2[1:5], o2[5:9], o2[9]
  num1, den1 = _run_edge(gl1_parts, gr1_parts, src, dst, att1sc, maxp1)
  hgsum, counts = _run_pool(h1, num1, den1, b1sc, batch32)
  return _run_final(hgsum, counts, Wp, bpr)
